# baseline (speedup 1.0000x reference)
import functools

import jax
import jax.numpy as jnp
from jax import lax
from jax.experimental import pallas as pl
from jax.experimental.pallas import tpu as pltpu
from jax.experimental.pallas import tpu_sc as plsc

TOKENS = 8192
D_IN = 1024
D_H = 512
NE = 16

NC = 2
NS = 16
L = 16
NW = NC * NS
TPW = TOKENS // NW
GROUPS = TPW // L


def _mlp_body(x_ref, w1_ref, b1_ref, w2_ref, b2_ref, out_ref):
    h = jax.lax.dot_general(
        x_ref[...], w1_ref[...], (((1,), (0,)), ((), ())),
        preferred_element_type=jnp.float32)
    h = jnp.maximum(h + b1_ref[...], 0.0)
    logits = jax.lax.dot_general(
        h, w2_ref[...], (((1,), (0,)), ((), ())),
        preferred_element_type=jnp.float32)
    out_ref[...] = logits + b2_ref[...]


def _mlp_logits(x, W1, b1, W2, b2, bm=2048):
    grid = (TOKENS // bm,)
    return pl.pallas_call(
        _mlp_body,
        grid=grid,
        in_specs=[
            pl.BlockSpec((bm, D_IN), lambda i: (i, 0)),
            pl.BlockSpec((D_IN, D_H), lambda i: (0, 0)),
            pl.BlockSpec((D_H,), lambda i: (0,)),
            pl.BlockSpec((D_H, NE), lambda i: (0, 0)),
            pl.BlockSpec((NE,), lambda i: (0,)),
        ],
        out_specs=pl.BlockSpec((bm, NE), lambda i: (i, 0)),
        out_shape=jax.ShapeDtypeStruct((TOKENS, NE), jnp.float32),
        compiler_params=pltpu.CompilerParams(
            dimension_semantics=("parallel",)),
    )(x, W1, b1, W2, b2)


def _make_gate_body(tpw):
    def _gate_body(logits_hbm, out_hbm, lbuf, obuf):
        wid = lax.axis_index("s") * NC + lax.axis_index("c")
        base = wid * tpw
        pltpu.sync_copy(logits_hbm.at[pl.ds(base, tpw)], lbuf)

        iota = lax.iota(jnp.int32, NE)

        def _one_token(t):
            l = lbuf[t, :]
            m1 = jnp.max(l, axis=0)
            i1 = plsc.all_reduce_ffs(l == m1)
            l2 = jnp.where(iota == i1, -3.0e38, l)
            m2 = jnp.max(l2, axis=0)
            i2 = plsc.all_reduce_ffs(l2 == m2)
            r = jnp.exp(jnp.broadcast_to(m2 - m1, (NE,)))
            v1 = 1.0 / (1.0 + r)
            v2 = 1.0 - v1
            obuf[t, :] = jnp.where(
                iota == i1, v1, jnp.where(iota == i2, v2, 0.0))

        unroll = 4
        def _group(g, carry):
            t0 = g * unroll
            for k in range(unroll):
                _one_token(t0 + k)
            return carry

        lax.fori_loop(0, tpw // unroll, _group, 0)

        pltpu.sync_copy(obuf, out_hbm.at[pl.ds(base, tpw)])

    return _gate_body


_GATE_SC = {}


def _gate_sc(logits):
    rows = logits.shape[0]
    if rows not in _GATE_SC:
        tpw = rows // NW
        _GATE_SC[rows] = functools.partial(
            pl.kernel,
            mesh=plsc.VectorSubcoreMesh(core_axis_name="c",
                                        subcore_axis_name="s"),
            out_type=jax.ShapeDtypeStruct((rows, NE), jnp.float32),
            scratch_types=[
                pltpu.VMEM((tpw, NE), jnp.float32),
                pltpu.VMEM((tpw, NE), jnp.float32),
            ],
            compiler_params=pltpu.CompilerParams(
                needs_layout_passes=False),
        )(_make_gate_body(tpw))
    return _GATE_SC[rows](logits)


def kernel(inputs, W1, b1, W2, b2):
    logits = _mlp_logits(inputs, W1, b1, W2, b2)
    return _gate_sc(logits)


# speedup vs baseline: 1.0045x; 1.0045x over previous
import functools

import jax
import jax.numpy as jnp
from jax import lax
from jax.experimental import pallas as pl
from jax.experimental.pallas import tpu as pltpu
from jax.experimental.pallas import tpu_sc as plsc

TOKENS = 8192
D_IN = 1024
D_H = 512
NE = 16

NC = 2
NS = 16
L = 16
NW = NC * NS
TPW = TOKENS // NW
GROUPS = TPW // L


def _mlp_body(x_ref, w1_ref, b1_ref, w2_ref, b2_ref, out_ref):
    h = jax.lax.dot_general(
        x_ref[...], w1_ref[...], (((1,), (0,)), ((), ())),
        preferred_element_type=jnp.float32)
    h = jnp.maximum(h + b1_ref[...], 0.0)
    logits = jax.lax.dot_general(
        h, w2_ref[...], (((1,), (0,)), ((), ())),
        preferred_element_type=jnp.float32)
    out_ref[...] = logits + b2_ref[...]


def _mlp_logits(x, W1, b1, W2, b2, bm=2048):
    grid = (TOKENS // bm,)
    return pl.pallas_call(
        _mlp_body,
        grid=grid,
        in_specs=[
            pl.BlockSpec((bm, D_IN), lambda i: (i, 0)),
            pl.BlockSpec((D_IN, D_H), lambda i: (0, 0)),
            pl.BlockSpec((D_H,), lambda i: (0,)),
            pl.BlockSpec((D_H, NE), lambda i: (0, 0)),
            pl.BlockSpec((NE,), lambda i: (0,)),
        ],
        out_specs=pl.BlockSpec((bm, NE), lambda i: (i, 0)),
        out_shape=jax.ShapeDtypeStruct((TOKENS, NE), jnp.float32),
        compiler_params=pltpu.CompilerParams(
            dimension_semantics=("parallel",)),
    )(x, W1, b1, W2, b2)


def _make_gate_body(tpw):
    def _gate_body(logits_hbm, out_hbm, lbuf, obuf):
        wid = lax.axis_index("s") * NC + lax.axis_index("c")
        base = wid * tpw
        pltpu.sync_copy(logits_hbm.at[pl.ds(base, tpw)], lbuf)

        iota = lax.iota(jnp.int32, NE)

        def _one_token(t):
            l = lbuf[t, :]
            m1 = jnp.max(l, axis=0)
            i1 = plsc.all_reduce_ffs(l == m1)
            l2 = jnp.where(iota == i1, -3.0e38, l)
            m2 = jnp.max(l2, axis=0)
            i2 = plsc.all_reduce_ffs(l2 == m2)
            r = jnp.exp(jnp.broadcast_to(m2 - m1, (NE,)))
            v1 = 1.0 / (1.0 + r)
            v2 = 1.0 - v1
            obuf[t, :] = jnp.where(
                iota == i1, v1, jnp.where(iota == i2, v2, 0.0))

        unroll = 8
        def _group(g, carry):
            t0 = g * unroll
            for k in range(unroll):
                _one_token(t0 + k)
            return carry

        lax.fori_loop(0, tpw // unroll, _group, 0)

        pltpu.sync_copy(obuf, out_hbm.at[pl.ds(base, tpw)])

    return _gate_body


_GATE_SC = {}


def _gate_sc(logits):
    rows = logits.shape[0]
    if rows not in _GATE_SC:
        tpw = rows // NW
        _GATE_SC[rows] = functools.partial(
            pl.kernel,
            mesh=plsc.VectorSubcoreMesh(core_axis_name="c",
                                        subcore_axis_name="s"),
            out_type=jax.ShapeDtypeStruct((rows, NE), jnp.float32),
            scratch_types=[
                pltpu.VMEM((tpw, NE), jnp.float32),
                pltpu.VMEM((tpw, NE), jnp.float32),
            ],
            compiler_params=pltpu.CompilerParams(
                needs_layout_passes=False),
        )(_make_gate_body(tpw))
    return _GATE_SC[rows](logits)


def kernel(inputs, W1, b1, W2, b2):
    logits = _mlp_logits(inputs, W1, b1, W2, b2)
    return _gate_sc(logits)


# speedup vs baseline: 1.0157x; 1.0111x over previous
import functools

import jax
import jax.numpy as jnp
from jax import lax
from jax.experimental import pallas as pl
from jax.experimental.pallas import tpu as pltpu
from jax.experimental.pallas import tpu_sc as plsc

TOKENS = 8192
D_IN = 1024
D_H = 512
NE = 16

NC = 2
NS = 16
L = 16
NW = NC * NS
TPW = TOKENS // NW
GROUPS = TPW // L


def _mlp_body(x_ref, w1_ref, b1_ref, w2_ref, b2_ref, out_ref):
    h = jax.lax.dot_general(
        x_ref[...], w1_ref[...], (((1,), (0,)), ((), ())),
        preferred_element_type=jnp.float32)
    h = jnp.maximum(h + b1_ref[...], 0.0)
    logits = jax.lax.dot_general(
        h, w2_ref[...], (((1,), (0,)), ((), ())),
        preferred_element_type=jnp.float32)
    out_ref[...] = logits + b2_ref[...]


def _mlp_logits(x, W1, b1, W2, b2, bm=2048):
    grid = (TOKENS // bm,)
    return pl.pallas_call(
        _mlp_body,
        grid=grid,
        in_specs=[
            pl.BlockSpec((bm, D_IN), lambda i: (i, 0)),
            pl.BlockSpec((D_IN, D_H), lambda i: (0, 0)),
            pl.BlockSpec((D_H,), lambda i: (0,)),
            pl.BlockSpec((D_H, NE), lambda i: (0, 0)),
            pl.BlockSpec((NE,), lambda i: (0,)),
        ],
        out_specs=pl.BlockSpec((bm, NE), lambda i: (i, 0)),
        out_shape=jax.ShapeDtypeStruct((TOKENS, NE), jnp.float32),
        compiler_params=pltpu.CompilerParams(
            dimension_semantics=("parallel",)),
    )(x, W1, b1, W2, b2)


def _make_gate_body(tpw):
    def _gate_body(logits_hbm, out_hbm, lbuf, obuf):
        wid = lax.axis_index("s") * NC + lax.axis_index("c")
        base = wid * tpw
        pltpu.sync_copy(logits_hbm.at[pl.ds(base, tpw)], lbuf)

        iota = lax.iota(jnp.int32, NE)

        def _one_token(t):
            l = lbuf[t, :]
            m1 = jnp.max(l, axis=0)
            i1 = plsc.all_reduce_ffs(l == m1)
            l2 = jnp.where(iota == i1, -3.0e38, l)
            m2 = jnp.max(l2, axis=0)
            i2 = plsc.all_reduce_ffs(l2 == m2)
            keep = (iota == i1) | (iota == i2)
            ek = jnp.where(keep, jnp.exp(l - m1), 0.0)
            denom = jnp.sum(ek, axis=0)
            obuf[t, :] = ek / jnp.broadcast_to(denom, (NE,))

        unroll = 4
        def _group(g, carry):
            t0 = g * unroll
            for k in range(unroll):
                _one_token(t0 + k)
            return carry

        lax.fori_loop(0, tpw // unroll, _group, 0)

        pltpu.sync_copy(obuf, out_hbm.at[pl.ds(base, tpw)])

    return _gate_body


_GATE_SC = {}


def _gate_sc(logits):
    rows = logits.shape[0]
    if rows not in _GATE_SC:
        tpw = rows // NW
        _GATE_SC[rows] = functools.partial(
            pl.kernel,
            mesh=plsc.VectorSubcoreMesh(core_axis_name="c",
                                        subcore_axis_name="s"),
            out_type=jax.ShapeDtypeStruct((rows, NE), jnp.float32),
            scratch_types=[
                pltpu.VMEM((tpw, NE), jnp.float32),
                pltpu.VMEM((tpw, NE), jnp.float32),
            ],
            compiler_params=pltpu.CompilerParams(
                needs_layout_passes=False),
        )(_make_gate_body(tpw))
    return _GATE_SC[rows](logits)


def kernel(inputs, W1, b1, W2, b2):
    logits = _mlp_logits(inputs, W1, b1, W2, b2)
    return _gate_sc(logits)
